# baseline (speedup 1.0000x reference)
import functools

import jax
import jax.numpy as jnp
from jax import lax
from jax.experimental import pallas as pl
from jax.experimental.pallas import tpu as pltpu
from jax.experimental.pallas import tpu_sc as plsc

VOCAB = 1000000
EMB = 64
SEQ = 200
BATCH = 4096

ROWS = 8000
NBLK = VOCAB // ROWS


def _proj_body(tab_ref, w_ref, out_ref):
    out_ref[...] = jnp.sum(tab_ref[...] * w_ref[...], axis=0)


def _proj_tc(table, W):
    tt = table.T.reshape(EMB, NBLK * 8, ROWS // 8)
    wt = W.reshape(EMB, 1, 1)
    return pl.pallas_call(
        _proj_body,
        grid=(NBLK,),
        in_specs=[
            pl.BlockSpec((EMB, 8, ROWS // 8), lambda g: (0, g, 0)),
            pl.BlockSpec((EMB, 1, 1), lambda g: (0, 0, 0)),
        ],
        out_specs=pl.BlockSpec((8, ROWS // 8), lambda g: (g, 0)),
        out_shape=jax.ShapeDtypeStruct((NBLK * 8, ROWS // 8), jnp.float32),
    )(tt, wt)


NC, NS, L = 2, 16, 16
NW = NC * NS
BPW = BATCH // NW
K = 25
NCH = SEQ // K
NLC = BPW // L

@functools.lru_cache(maxsize=1)
def _make_pool_sc():
    mesh = plsc.VectorSubcoreMesh(
        core_axis_name="c", subcore_axis_name="s",
        num_cores=NC, num_subcores=NS)
    return pl.kernel(
        _pool_sc_body,
        mesh=mesh,
        out_type=jax.ShapeDtypeStruct((BATCH,), jnp.float32),
        scratch_types=[
            pltpu.VMEM((SEQ, BPW), jnp.int32),
            pltpu.VMEM((SEQ, BPW), jnp.float32),
            pltpu.VMEM((BPW,), jnp.float32),
            pltpu.VMEM((L,), jnp.float32),
            pltpu.SemaphoreType.DMA,
            pltpu.SemaphoreType.DMA,
        ],
    )


def _pool_sc_body(x_hbm, proj_hbm, b_hbm, out_hbm, idx_v, vals_v, y_v, b_v, sem0, sem1):
    wid = lax.axis_index("s") * NC + lax.axis_index("c")
    base = wid * BPW
    pltpu.sync_copy(b_hbm, b_v)
    pltpu.sync_copy(x_hbm.at[:, pl.ds(base, BPW)], idx_v)

    def fire(c0, sem):
        def body(s, carry):
            pltpu.make_async_copy(
                proj_hbm.at[idx_v.at[s]], vals_v.at[s], sem).start()
            return carry
        lax.fori_loop(c0, c0 + K, body, 0)

    def drain(c0, sem):
        def body(s, carry):
            pltpu.make_async_copy(
                proj_hbm.at[idx_v.at[s]], vals_v.at[s], sem).wait()
            return carry
        lax.fori_loop(c0, c0 + K, body, 0)

    def accumulate(c0, accs):
        def body(s, accs):
            return tuple(accs[j] + vals_v[s, pl.ds(j * L, L)]
                         for j in range(NLC))
        return lax.fori_loop(c0, c0 + K, body, accs)

    sems = (sem0, sem1)
    accs = tuple(jnp.zeros((L,), jnp.float32) for _ in range(NLC))
    fire(0, sems[0])
    for i in range(NCH):
        if i + 1 < NCH:
            fire((i + 1) * K, sems[(i + 1) % 2])
        drain(i * K, sems[i % 2])
        accs = accumulate(i * K, accs)

    bvec = b_v[...]
    for j in range(NLC):
        z = accs[j] * (1.0 / SEQ) + bvec
        y_v[pl.ds(j * L, L)] = 1.0 / (1.0 + jnp.exp(-z))
    pltpu.sync_copy(y_v, out_hbm.at[pl.ds(base, BPW)])


def kernel(x, table, W, b):
    proj = _proj_tc(table, W).reshape(VOCAB)
    b16 = jnp.broadcast_to(b.astype(jnp.float32), (L,))
    return _make_pool_sc()(x, proj, b16)


# speedup vs baseline: 3.1888x; 3.1888x over previous
import functools

import jax
import jax.numpy as jnp
from jax import lax
from jax.experimental import pallas as pl
from jax.experimental.pallas import tpu as pltpu
from jax.experimental.pallas import tpu_sc as plsc

VOCAB = 1000000
EMB = 64
SEQ = 200
BATCH = 4096

GRP = EMB // 8
CH = 65536


def _proj_acc_body(tt_hbm, w_ref, out_ref, buf, sem):
    for g in range(GRP):
        cp = pltpu.make_async_copy(tt_hbm.at[pl.ds(8 * g, 8), :], buf, sem)
        cp.start()
        cp.wait()
        w = w_ref[:, pl.ds(g * 128, 128)][:, :1]
        for c in range(0, VOCAB, CH):
            n = min(CH, VOCAB - c)
            p = buf[:, pl.ds(c, n)] * w
            if g == 0:
                out_ref[:, pl.ds(c, n)] = p
            else:
                out_ref[:, pl.ds(c, n)] += p


def _proj_red_body(acc_hbm, out_ref, buf, sem):
    cp = pltpu.make_async_copy(acc_hbm, buf, sem)
    cp.start()
    cp.wait()
    for c in range(0, VOCAB, CH):
        n = min(CH, VOCAB - c)
        out_ref[pl.ds(c, n)] = jnp.sum(buf[:, pl.ds(c, n)], axis=0)


def _proj_tc(table, W):
    tt = table.T
    warr = jnp.repeat(W.reshape(GRP, 8).T, 128, axis=1)
    acc = pl.pallas_call(
        _proj_acc_body,
        in_specs=[
            pl.BlockSpec(memory_space=pl.ANY),
            pl.BlockSpec(memory_space=pltpu.VMEM),
        ],
        out_specs=pl.BlockSpec(memory_space=pltpu.VMEM),
        out_shape=jax.ShapeDtypeStruct((8, VOCAB), jnp.float32),
        scratch_shapes=[
            pltpu.VMEM((8, VOCAB), jnp.float32),
            pltpu.SemaphoreType.DMA,
        ],
        compiler_params=pltpu.CompilerParams(
            vmem_limit_bytes=63 * 1024 * 1024,
        ),
    )(tt, warr)
    return pl.pallas_call(
        _proj_red_body,
        in_specs=[pl.BlockSpec(memory_space=pl.ANY)],
        out_specs=pl.BlockSpec(memory_space=pltpu.VMEM),
        out_shape=jax.ShapeDtypeStruct((VOCAB,), jnp.float32),
        scratch_shapes=[
            pltpu.VMEM((8, VOCAB), jnp.float32),
            pltpu.SemaphoreType.DMA,
        ],
        compiler_params=pltpu.CompilerParams(
            vmem_limit_bytes=63 * 1024 * 1024,
        ),
    )(acc)


NC, NS, L = 2, 16, 16
NW = NC * NS
BPW = BATCH // NW
K = 25
NCH = SEQ // K
NLC = BPW // L

@functools.lru_cache(maxsize=1)
def _make_pool_sc():
    mesh = plsc.VectorSubcoreMesh(
        core_axis_name="c", subcore_axis_name="s",
        num_cores=NC, num_subcores=NS)
    return pl.kernel(
        _pool_sc_body,
        mesh=mesh,
        out_type=jax.ShapeDtypeStruct((BATCH,), jnp.float32),
        scratch_types=[
            pltpu.VMEM((SEQ, BPW), jnp.int32),
            pltpu.VMEM((SEQ, BPW), jnp.float32),
            pltpu.VMEM((BPW,), jnp.float32),
            pltpu.VMEM((L,), jnp.float32),
            pltpu.SemaphoreType.DMA,
            pltpu.SemaphoreType.DMA,
        ],
    )


def _pool_sc_body(x_hbm, proj_hbm, b_hbm, out_hbm, idx_v, vals_v, y_v, b_v, sem0, sem1):
    wid = lax.axis_index("s") * NC + lax.axis_index("c")
    base = wid * BPW
    pltpu.sync_copy(b_hbm, b_v)
    pltpu.sync_copy(x_hbm.at[:, pl.ds(base, BPW)], idx_v)

    def fire(c0, sem):
        def body(s, carry):
            pltpu.make_async_copy(
                proj_hbm.at[idx_v.at[s]], vals_v.at[s], sem).start()
            return carry
        lax.fori_loop(c0, c0 + K, body, 0)

    def drain(c0, sem):
        def body(s, carry):
            pltpu.make_async_copy(
                proj_hbm.at[idx_v.at[s]], vals_v.at[s], sem).wait()
            return carry
        lax.fori_loop(c0, c0 + K, body, 0)

    def accumulate(c0, accs):
        def body(s, accs):
            return tuple(accs[j] + vals_v[s, pl.ds(j * L, L)]
                         for j in range(NLC))
        return lax.fori_loop(c0, c0 + K, body, accs)

    sems = (sem0, sem1)
    accs = tuple(jnp.zeros((L,), jnp.float32) for _ in range(NLC))
    fire(0, sems[0])
    for i in range(NCH):
        if i + 1 < NCH:
            fire((i + 1) * K, sems[(i + 1) % 2])
        drain(i * K, sems[i % 2])
        accs = accumulate(i * K, accs)

    bvec = b_v[...]
    for j in range(NLC):
        z = accs[j] * (1.0 / SEQ) + bvec
        y_v[pl.ds(j * L, L)] = 1.0 / (1.0 + jnp.exp(-z))
    pltpu.sync_copy(y_v, out_hbm.at[pl.ds(base, BPW)])


def kernel(x, table, W, b):
    proj = _proj_tc(table, W)
    b16 = jnp.broadcast_to(b.astype(jnp.float32), (L,))
    return _make_pool_sc()(x, proj, b16)


# speedup vs baseline: 3.9629x; 1.2427x over previous
import functools

import jax
import jax.numpy as jnp
from jax import lax
from jax.experimental import pallas as pl
from jax.experimental.pallas import tpu as pltpu
from jax.experimental.pallas import tpu_sc as plsc

VOCAB = 1000000
EMB = 64
SEQ = 200
BATCH = 4096

GRP = EMB // 8
CH = 65536


HALF0 = 499968
HALF1 = VOCAB - HALF0
_HOFF = (0, HALF0)
_HLEN = (HALF0, HALF1)


def _proj_acc_body(tt_hbm, w_ref, out_ref, buf0, buf1, sem0, sem1):
    bufs, sems = (buf0, buf1), (sem0, sem1)

    def copy(h):
        g, side = divmod(h, 2)
        return pltpu.make_async_copy(
            tt_hbm.at[pl.ds(8 * g, 8), pl.ds(_HOFF[side], _HLEN[side])],
            bufs[h % 2].at[:, pl.ds(0, _HLEN[side])], sems[h % 2])

    copy(0).start()
    copy(1).start()
    for h in range(2 * GRP):
        g, side = divmod(h, 2)
        copy(h).wait()
        w = w_ref[:, pl.ds(g * 128, 128)][:, :1]
        buf = bufs[h % 2]
        for c in range(0, _HLEN[side], CH):
            n = min(CH, _HLEN[side] - c)
            p = buf[:, pl.ds(c, n)] * w
            if g == 0:
                out_ref[:, pl.ds(_HOFF[side] + c, n)] = p
            else:
                out_ref[:, pl.ds(_HOFF[side] + c, n)] += p
        if h + 2 < 2 * GRP:
            copy(h + 2).start()


def _proj_red_body(acc_hbm, out_ref, buf, sem):
    cp = pltpu.make_async_copy(acc_hbm, buf, sem)
    cp.start()
    cp.wait()
    for c in range(0, VOCAB, CH):
        n = min(CH, VOCAB - c)
        out_ref[pl.ds(c, n)] = jnp.sum(buf[:, pl.ds(c, n)], axis=0)


def _proj_tc(table, W):
    tt = table.T
    warr = jnp.repeat(W.reshape(GRP, 8).T, 128, axis=1)
    acc = pl.pallas_call(
        _proj_acc_body,
        in_specs=[
            pl.BlockSpec(memory_space=pl.ANY),
            pl.BlockSpec(memory_space=pltpu.VMEM),
        ],
        out_specs=pl.BlockSpec(memory_space=pltpu.VMEM),
        out_shape=jax.ShapeDtypeStruct((8, VOCAB), jnp.float32),
        scratch_shapes=[
            pltpu.VMEM((8, HALF1), jnp.float32),
            pltpu.VMEM((8, HALF1), jnp.float32),
            pltpu.SemaphoreType.DMA,
            pltpu.SemaphoreType.DMA,
        ],
        compiler_params=pltpu.CompilerParams(
            vmem_limit_bytes=63 * 1024 * 1024,
        ),
    )(tt, warr)
    return pl.pallas_call(
        _proj_red_body,
        in_specs=[pl.BlockSpec(memory_space=pl.ANY)],
        out_specs=pl.BlockSpec(memory_space=pltpu.VMEM),
        out_shape=jax.ShapeDtypeStruct((VOCAB,), jnp.float32),
        scratch_shapes=[
            pltpu.VMEM((8, VOCAB), jnp.float32),
            pltpu.SemaphoreType.DMA,
        ],
        compiler_params=pltpu.CompilerParams(
            vmem_limit_bytes=63 * 1024 * 1024,
        ),
    )(acc)


NC, NS, L = 2, 16, 16
NW = NC * NS
BPW = BATCH // NW
K = 25
NCH = SEQ // K
NLC = BPW // L

@functools.lru_cache(maxsize=1)
def _make_pool_sc():
    mesh = plsc.VectorSubcoreMesh(
        core_axis_name="c", subcore_axis_name="s",
        num_cores=NC, num_subcores=NS)
    return pl.kernel(
        _pool_sc_body,
        mesh=mesh,
        out_type=jax.ShapeDtypeStruct((BATCH,), jnp.float32),
        scratch_types=[
            pltpu.VMEM((SEQ, BPW), jnp.int32),
            pltpu.VMEM((SEQ, BPW), jnp.float32),
            pltpu.VMEM((BPW,), jnp.float32),
            pltpu.VMEM((L,), jnp.float32),
            pltpu.SemaphoreType.DMA,
            pltpu.SemaphoreType.DMA,
        ],
    )


def _pool_sc_body(x_hbm, proj_hbm, b_hbm, out_hbm, idx_v, vals_v, y_v, b_v, sem0, sem1):
    wid = lax.axis_index("s") * NC + lax.axis_index("c")
    base = wid * BPW
    pltpu.sync_copy(b_hbm, b_v)
    pltpu.sync_copy(x_hbm.at[:, pl.ds(base, BPW)], idx_v)

    def fire(c0, sem):
        def body(s, carry):
            pltpu.make_async_copy(
                proj_hbm.at[idx_v.at[s]], vals_v.at[s], sem).start()
            return carry
        lax.fori_loop(c0, c0 + K, body, 0)

    def drain(c0, sem):
        def body(s, carry):
            pltpu.make_async_copy(
                proj_hbm.at[idx_v.at[s]], vals_v.at[s], sem).wait()
            return carry
        lax.fori_loop(c0, c0 + K, body, 0)

    def accumulate(c0, accs):
        def body(s, accs):
            return tuple(accs[j] + vals_v[s, pl.ds(j * L, L)]
                         for j in range(NLC))
        return lax.fori_loop(c0, c0 + K, body, accs)

    sems = (sem0, sem1)
    accs = tuple(jnp.zeros((L,), jnp.float32) for _ in range(NLC))
    fire(0, sems[0])
    for i in range(NCH):
        if i + 1 < NCH:
            fire((i + 1) * K, sems[(i + 1) % 2])
        drain(i * K, sems[i % 2])
        accs = accumulate(i * K, accs)

    bvec = b_v[...]
    for j in range(NLC):
        z = accs[j] * (1.0 / SEQ) + bvec
        y_v[pl.ds(j * L, L)] = 1.0 / (1.0 + jnp.exp(-z))
    pltpu.sync_copy(y_v, out_hbm.at[pl.ds(base, BPW)])


def kernel(x, table, W, b):
    proj = _proj_tc(table, W)
    b16 = jnp.broadcast_to(b.astype(jnp.float32), (L,))
    return _make_pool_sc()(x, proj, b16)
